# Initial kernel scaffold; baseline (speedup 1.0000x reference)
import jax
import jax.numpy as jnp
from jax.experimental import pallas as pl


def kernel(x, w_fc, b_fc, w_conv, b_conv, bn_gamma, bn_beta, bn_mean, bn_var):
    raise NotImplementedError("write your pallas kernel here")


# speedup vs baseline: 3.7447x; 3.7447x over previous
import functools
import numpy as np

import jax
import jax.numpy as jnp
from jax.experimental import pallas as pl
from jax.experimental.pallas import tpu as pltpu


def _graph_kernel(x_ref, w_ref, b_ref, o_ref, *, P, K):
    xs = [x_ref[p] for p in range(P)]

    ss = [jnp.sum(xs[p] * xs[p], axis=0, keepdims=True) for p in range(P)]
    pd = [[None] * P for _ in range(P)]
    for p in range(P):
        for q in range(p, P):
            gpq = jnp.sum(xs[p] * xs[q], axis=0, keepdims=True)
            d = 2.0 * gpq - ss[p] - ss[q]
            pd[p][q] = d
            pd[q][p] = d

    rank = [[None] * P for _ in range(P)]
    for p in range(P):
        for q in range(P):
            r = None
            for qp in range(P):
                if qp == q:
                    continue
                if qp < q:
                    b = (pd[p][qp] >= pd[p][q])
                else:
                    b = (pd[p][qp] > pd[p][q])
                b = b.astype(jnp.float32)
                r = b if r is None else r + b
            rank[p][q] = r

    inv = [jax.lax.rsqrt(jnp.maximum(ss[p], 1e-24)) for p in range(P)]

    for p in range(P):
        coefs = [[None] * P for _ in range(K)]
        for q in range(P):
            r = rank[p][q]
            for i in range(K):
                c = w_ref[i * K + 0] * (r == 0.0).astype(jnp.float32)
                for j in range(1, K):
                    c = c + w_ref[i * K + j] * (r == jnp.float32(j)).astype(jnp.float32)
                coefs[i][q] = c * inv[q]
        out_p = None
        for i in range(K):
            acc = None
            for q in range(P):
                term = xs[q] * coefs[i][q]
                acc = term if acc is None else acc + term
            yi = jnp.maximum(acc + b_ref[i], 0.0)
            out_p = yi if out_p is None else jnp.maximum(out_p, yi)
        o_ref[p] = out_p


def _graph_stage(xpc, w_fc, b_fc, K, T):
    P, C, BN = xpc.shape
    BNp = ((BN + T - 1) // T) * T
    if BNp != BN:
        xpc = jnp.pad(xpc, ((0, 0), (0, 0), (0, BNp - BN)))
    out = pl.pallas_call(
        functools.partial(_graph_kernel, P=P, K=K),
        out_shape=jax.ShapeDtypeStruct((P, C, BNp), jnp.float32),
        grid=(BNp // T,),
        in_specs=[
            pl.BlockSpec((P, C, T), lambda i: (0, 0, i)),
            pl.BlockSpec(memory_space=pltpu.MemorySpace.SMEM),
            pl.BlockSpec(memory_space=pltpu.MemorySpace.SMEM),
        ],
        out_specs=pl.BlockSpec((P, C, T), lambda i: (0, 0, i)),
        compiler_params=pltpu.CompilerParams(dimension_semantics=("parallel",)),
    )(xpc, w_fc.reshape(-1).astype(jnp.float32), b_fc.astype(jnp.float32))
    return out[:, :, :BN]


def _conv_pool_kernel(m_ref, w_ref, t_ref, mask_ref, o_ref, col_ref, *, W, pool):
    m = m_ref[0]
    C, HW = m.shape
    for dy in (-1, 0, 1):
        for dx in (-1, 0, 1):
            tap = (dy + 1) * 3 + (dx + 1)
            s = dy * W + dx
            shifted = m if s == 0 else pltpu.roll(m, (-s) % HW, axis=1)
            if not (dy == 0 and dx == 0):
                shifted = shifted * mask_ref[tap:tap + 1, :]
            col_ref[tap * C:(tap + 1) * C, :] = shifted.astype(jnp.bfloat16)

    z = jnp.dot(w_ref[...], col_ref[...], preferred_element_type=jnp.float32)
    z = jnp.maximum(z + t_ref[...], 0.0)

    pp = pool // 2
    out = z
    for dy in range(-pp, pp + 1):
        for dx in range(-pp, pp + 1):
            if dy == 0 and dx == 0:
                continue
            s = dy * W + dx
            shifted = pltpu.roll(z, (-s) % HW, axis=1)
            j = (dy + pp) * pool + (dx + pp)
            out = jnp.maximum(out, shifted * mask_ref[j:j + 1, :])
    o_ref[0] = out


def _conv_stage(m, w_conv, b_conv, gamma, beta, rmean, rvar, pool):
    B, C, H, W = m.shape
    O = w_conv.shape[0]
    HW = H * W
    pp = pool // 2
    eps = 1e-5

    s = gamma / jnp.sqrt(rvar + eps)
    t = ((b_conv - rmean) * s + beta).reshape(O, 1)
    w_eff = ((w_conv * s[:, None, None, None])
             .transpose(0, 2, 3, 1).reshape(O, 9 * C).astype(jnp.bfloat16))

    ys, xs_ = np.arange(HW) // W, np.arange(HW) % W
    masks = np.zeros((pool * pool, HW), np.float32)
    for dy in range(-pp, pp + 1):
        for dx in range(-pp, pp + 1):
            ok = (ys + dy >= 0) & (ys + dy < H) & (xs_ + dx >= 0) & (xs_ + dx < W)
            masks[(dy + pp) * pool + (dx + pp)] = ok
    masks = jnp.asarray(masks)

    out = pl.pallas_call(
        functools.partial(_conv_pool_kernel, W=W, pool=pool),
        out_shape=jax.ShapeDtypeStruct((B, O, HW), jnp.float32),
        grid=(B,),
        in_specs=[
            pl.BlockSpec((1, C, HW), lambda b: (b, 0, 0)),
            pl.BlockSpec((O, 9 * C), lambda b: (0, 0)),
            pl.BlockSpec((O, 1), lambda b: (0, 0)),
            pl.BlockSpec((pool * pool, HW), lambda b: (0, 0)),
        ],
        out_specs=pl.BlockSpec((1, O, HW), lambda b: (b, 0, 0)),
        scratch_shapes=[pltpu.VMEM((9 * C, HW), jnp.bfloat16)],
        compiler_params=pltpu.CompilerParams(dimension_semantics=("parallel",)),
    )(m.reshape(B, C, HW), w_eff, t, masks)
    return out.reshape(B, O, H, W)


def kernel(x, w_fc, b_fc, w_conv, b_conv, bn_gamma, bn_beta, bn_mean, bn_var):
    B, C, H, W = x.shape
    ws, k, pool = 2, 4, 3
    pad = ws // 2

    xp = jnp.pad(x, ((0, 0), (0, 0), (pad, pad), (pad, pad)))
    nH = (H + 2 * pad) // ws
    nW = (W + 2 * pad) // ws
    xpc = (xp.reshape(B, C, nH, ws, nW, ws)
           .transpose(3, 5, 1, 0, 2, 4)
           .reshape(ws * ws, C, B * nH * nW))

    g = _graph_stage(xpc, w_fc, b_fc, k, T=256)

    g = (g.reshape(ws, ws, C, B, nH, nW)
         .transpose(3, 2, 4, 0, 5, 1)
         .reshape(B, C, nH * ws, nW * ws))
    m = g[:, :, pad:pad + H, pad:pad + W]

    return _conv_stage(m, w_conv, b_conv, bn_gamma, bn_beta, bn_mean,
                       bn_var, pool)


# speedup vs baseline: 5.4397x; 1.4526x over previous
import functools
import numpy as np

import jax
import jax.numpy as jnp
from jax.experimental import pallas as pl
from jax.experimental.pallas import tpu as pltpu


def _shift(a, s, hw):
    return a if s == 0 else pltpu.roll(a, (-s) % hw, axis=a.ndim - 1)


def _fused_kernel(x_ref, w_ref, b_ref, wc_ref, t_ref, c_ref, o_ref, col_ref,
                  *, W, K):
    m = x_ref[0]
    C, HW = m.shape
    one = jnp.float32(1.0)

    maskH = c_ref[0:1, :]
    maskV = c_ref[1:2, :]
    maskD = c_ref[2:3, :]
    selR = c_ref[3:4, :]
    selD = c_ref[4:5, :]
    tbHS = c_ref[5:6, :]
    tbVS = c_ref[6:7, :]
    tbDS = c_ref[7:8, :]
    tbHV = c_ref[8:9, :]
    tbHD = c_ref[9:10, :]
    tbVD = c_ref[10:11, :]

    rp1, rm1 = _shift(m, 1, HW), _shift(m, -1, HW)
    vH = (selR * rp1 + (one - selR) * rm1) * maskH
    rpW, rmW = _shift(m, W, HW), _shift(m, -W, HW)
    vV = (selD * rpW + (one - selD) * rmW) * maskV
    rpp = _shift(m, W + 1, HW)
    rpm = _shift(m, W - 1, HW)
    rmp = _shift(m, -(W - 1), HW)
    rmm = _shift(m, -(W + 1), HW)
    sRD = selR * selD
    vD = (sRD * rpp + (selD - sRD) * rpm + (selR - sRD) * rmp
          + (one - selD - selR + sRD) * rmm) * maskD

    ss = jnp.sum(m * m, axis=0, keepdims=True)
    ssH = (selR * _shift(ss, 1, HW) + (one - selR) * _shift(ss, -1, HW)) * maskH
    ssV = (selD * _shift(ss, W, HW) + (one - selD) * _shift(ss, -W, HW)) * maskV
    ssD = (sRD * _shift(ss, W + 1, HW) + (selD - sRD) * _shift(ss, W - 1, HW)
           + (selR - sRD) * _shift(ss, -(W - 1), HW)
           + (one - selD - selR + sRD) * _shift(ss, -(W + 1), HW)) * maskD
    pdH = 2.0 * jnp.sum(m * vH, axis=0, keepdims=True) - ss - ssH
    pdV = 2.0 * jnp.sum(m * vV, axis=0, keepdims=True) - ss - ssV
    pdD = 2.0 * jnp.sum(m * vD, axis=0, keepdims=True) - ss - ssD
    pdS = jnp.zeros_like(ss)

    def bet(pa, pq, tb):
        return ((pa > pq) | ((pa == pq) & (tb > 0.5))).astype(jnp.float32)

    bHS = bet(pdH, pdS, tbHS)
    bVS = bet(pdV, pdS, tbVS)
    bDS = bet(pdD, pdS, tbDS)
    bHV = bet(pdH, pdV, tbHV)
    bHD = bet(pdH, pdD, tbHD)
    bVD = bet(pdV, pdD, tbVD)
    rankS = bHS + bVS + bDS
    rankH = (one - bHS) + (one - bHV) + (one - bHD)
    rankV = (one - bVS) + bHV + (one - bVD)
    rankD = (one - bDS) + bHD + bVD

    invS = jax.lax.rsqrt(jnp.maximum(ss, 1e-24))
    invH = jax.lax.rsqrt(jnp.maximum(ssH, 1e-24))
    invV = jax.lax.rsqrt(jnp.maximum(ssV, 1e-24))
    invD = jax.lax.rsqrt(jnp.maximum(ssD, 1e-24))

    def coef(rank, inv, i):
        c = w_ref[i * K + 0] * (rank == 0.0).astype(jnp.float32)
        for j in range(1, K):
            c = c + w_ref[i * K + j] * (rank == jnp.float32(j)).astype(jnp.float32)
        return c * inv

    gout = None
    for i in range(K):
        pre = (coef(rankS, invS, i) * m + coef(rankH, invH, i) * vH
               + coef(rankV, invV, i) * vV + coef(rankD, invD, i) * vD)
        yi = jnp.maximum(pre + b_ref[i], 0.0)
        gout = yi if gout is None else jnp.maximum(gout, yi)

    for dy in (-1, 0, 1):
        for dx in (-1, 0, 1):
            tap = (dy + 1) * 3 + (dx + 1)
            shifted = _shift(gout, dy * W + dx, HW)
            if not (dy == 0 and dx == 0):
                shifted = shifted * c_ref[11 + tap:12 + tap, :]
            col_ref[tap * C:(tap + 1) * C, :] = shifted.astype(jnp.bfloat16)

    z = jnp.dot(wc_ref[...], col_ref[...], preferred_element_type=jnp.float32)
    z = jnp.maximum(z + t_ref[...], 0.0)

    out = z
    for dy in (-1, 0, 1):
        for dx in (-1, 0, 1):
            if dy == 0 and dx == 0:
                continue
            tap = (dy + 1) * 3 + (dx + 1)
            shifted = _shift(z, dy * W + dx, HW)
            out = jnp.maximum(out, shifted * c_ref[11 + tap:12 + tap, :])
    o_ref[0] = out


def _build_consts(H, W):
    HW = H * W
    ys, xs = np.arange(HW) // W, np.arange(HW) % W
    xodd = (xs % 2 == 1)
    yodd = (ys % 2 == 1)
    c = np.zeros((20, HW), np.float32)
    c[0] = np.where(xodd, xs + 1 < W, xs - 1 >= 0)
    c[1] = np.where(yodd, ys + 1 < H, ys - 1 >= 0)
    c[2] = c[0] * c[1]
    c[3] = xodd
    c[4] = yodd
    c[5] = ~xodd
    c[6] = ~yodd
    c[7] = ~yodd
    c[8] = yodd
    c[9] = yodd
    c[10] = xodd
    for dy in (-1, 0, 1):
        for dx in (-1, 0, 1):
            ok = (ys + dy >= 0) & (ys + dy < H) & (xs + dx >= 0) & (xs + dx < W)
            c[11 + (dy + 1) * 3 + (dx + 1)] = ok
    return jnp.asarray(c)


def kernel(x, w_fc, b_fc, w_conv, b_conv, bn_gamma, bn_beta, bn_mean, bn_var):
    B, C, H, W = x.shape
    O = w_conv.shape[0]
    HW = H * W
    K = 4
    eps = 1e-5

    s = bn_gamma / jnp.sqrt(bn_var + eps)
    t = ((b_conv - bn_mean) * s + bn_beta).reshape(O, 1)
    w_eff = ((w_conv * s[:, None, None, None])
             .transpose(0, 2, 3, 1).reshape(O, 9 * C).astype(jnp.bfloat16))
    consts = _build_consts(H, W)

    out = pl.pallas_call(
        functools.partial(_fused_kernel, W=W, K=K),
        out_shape=jax.ShapeDtypeStruct((B, O, HW), jnp.float32),
        grid=(B,),
        in_specs=[
            pl.BlockSpec((1, C, HW), lambda b: (b, 0, 0)),
            pl.BlockSpec(memory_space=pltpu.MemorySpace.SMEM),
            pl.BlockSpec(memory_space=pltpu.MemorySpace.SMEM),
            pl.BlockSpec((O, 9 * C), lambda b: (0, 0)),
            pl.BlockSpec((O, 1), lambda b: (0, 0)),
            pl.BlockSpec((20, HW), lambda b: (0, 0)),
        ],
        out_specs=pl.BlockSpec((1, O, HW), lambda b: (b, 0, 0)),
        scratch_shapes=[pltpu.VMEM((9 * C, HW), jnp.bfloat16)],
        compiler_params=pltpu.CompilerParams(dimension_semantics=("parallel",)),
    )(x.reshape(B, C, HW), w_fc.reshape(-1).astype(jnp.float32),
      b_fc.astype(jnp.float32), w_eff, t, consts)
    return out.reshape(B, O, H, W)
